# baseline (speedup 1.0000x reference)
import functools

import jax
import jax.numpy as jnp
from jax import lax
from jax.experimental import pallas as pl
from jax.experimental.pallas import tpu as pltpu
from jax.experimental.pallas import tpu_sc as plsc

_HB = 4
_NB = 2


@functools.lru_cache(maxsize=None)
def _make_gather(bb, hh, d):
    info = plsc.get_sparse_core_info()
    nc, ns, nl = info.num_cores, info.num_subcores, info.num_lanes
    nw = nc * ns
    n = bb * hh
    hb = _HB
    nb = _NB
    cpw = 4 * (hh // hb)
    rows = hb * 128
    per_w = n // nw
    assert bb == nw * 4 * 128 and hh % hb == 0 and d % 8 == 0

    mesh = plsc.VectorSubcoreMesh(core_axis_name="c", subcore_axis_name="s")

    @functools.partial(
        pl.kernel,
        mesh=mesh,
        out_type=jax.ShapeDtypeStruct((hh, d // 8, bb // 128, 8, 128),
                                      jnp.float32),
        compiler_params=pltpu.CompilerParams(use_tc_tiling_on_sc=False,
                                             needs_layout_passes=False),
        scratch_types=[
            pltpu.VMEM((nb, rows), jnp.int32),
            pltpu.VMEM((nb, rows, d), jnp.float32),
            pltpu.VMEM((nb, hb, d // 8, 8, 128), jnp.float32),
        ] + [pltpu.SemaphoreType.DMA] * (2 * nb),
    )
    def gather_kernel(idx_hbm, table_hbm, out_hbm, idx_v, g_v, t_v, *sems):
        sem_g, sem_s = sems[:nb], sems[nb:]
        wid = lax.axis_index("s") * nc + lax.axis_index("c")
        base = wid * per_w
        iota = lax.iota(jnp.int32, nl)

        def issue_gather(i, b):
            off = pl.multiple_of(base + i * rows, 8)
            pltpu.sync_copy(idx_hbm.at[pl.ds(off, rows)], idx_v.at[b])
            pltpu.async_copy(table_hbm.at[idx_v.at[b]], g_v.at[b], sem_g[b])

        def wait_gather(b):
            pltpu.make_async_copy(
                table_hbm.at[idx_v.at[b]], g_v.at[b], sem_g[b]).wait()

        def out_slice(i):
            ct = wid * 4 + i // (hh // hb)
            h0 = (i % (hh // hb)) * hb
            return out_hbm.at[pl.ds(h0, hb), :, ct]

        def issue_store(i, b):
            pltpu.async_copy(t_v.at[b], out_slice(i), sem_s[b])

        def wait_store(i, b):
            pltpu.make_async_copy(t_v.at[b], out_slice(i), sem_s[b]).wait()

        def transpose(b):
            g2, t2 = g_v.at[b], t_v.at[b]

            def body(k, carry):
                c16 = pl.multiple_of(k * nl, nl)
                for hp in range(hb):
                    rvec = iota + (hp * 128 + c16)
                    for dd in range(d):
                        cvec = jnp.full((nl,), dd, jnp.int32)
                        vec = plsc.load_gather(g2, [rvec, cvec])
                        t2[hp, dd // 8, dd % 8, pl.ds(c16, nl)] = vec
                return carry

            lax.fori_loop(0, 128 // nl, body, 0, unroll=False)

        for b in range(nb):
            issue_gather(b, b)

        def outer(jo, carry):
            for b in range(nb):
                i = jo * nb + b
                wait_gather(b)
                transpose(b)
                issue_store(i, b)
                wait_store(i, b)
                issue_gather(i + nb, b)
            return carry

        lax.fori_loop(0, cpw // nb - 1, outer, 0, unroll=False)

        for b in range(nb):
            i = cpw - nb + b
            wait_gather(b)
            transpose(b)
            issue_store(i, b)
            wait_store(i, b)

    return gather_kernel


def kernel(x, table):
    b, h = x.shape
    _, d = table.shape
    nw = 32
    hb = _HB
    xp = (x.astype(jnp.int32)
          .reshape(nw, 4, 128, h // hb, hb)
          .transpose(0, 1, 3, 4, 2)
          .reshape(b * h))
    u = _make_gather(b, h, d)(xp, table)
    return u.transpose(2, 4, 0, 1, 3).reshape(b, h, d)


# speedup vs baseline: 2.0860x; 2.0860x over previous
import functools

import jax
import jax.numpy as jnp
from jax import lax
from jax.experimental import pallas as pl
from jax.experimental.pallas import tpu as pltpu
from jax.experimental.pallas import tpu_sc as plsc

_HB = 4
_NB = 2


@functools.lru_cache(maxsize=None)
def _make_gather(bb, hh, d):
    info = plsc.get_sparse_core_info()
    nc, ns, nl = info.num_cores, info.num_subcores, info.num_lanes
    nw = nc * ns
    n = bb * hh
    hb = _HB
    nb = _NB
    cpw = 4 * (hh // hb)
    rows = hb * 128
    per_w = n // nw
    assert bb == nw * 4 * 128 and hh % hb == 0 and d % 8 == 0

    mesh = plsc.VectorSubcoreMesh(core_axis_name="c", subcore_axis_name="s")

    @functools.partial(
        pl.kernel,
        mesh=mesh,
        out_type=jax.ShapeDtypeStruct((hh, d // 8, bb // 128, 8, 128),
                                      jnp.float32),
        compiler_params=pltpu.CompilerParams(use_tc_tiling_on_sc=False,
                                             needs_layout_passes=False),
        scratch_types=[
            pltpu.VMEM((nb, rows), jnp.int32),
            pltpu.VMEM((nb, rows, d), jnp.float32),
            pltpu.VMEM((nb, hb, d // 8, 8, 128), jnp.float32),
        ] + [pltpu.SemaphoreType.DMA] * (2 * nb),
    )
    def gather_kernel(idx_hbm, table_hbm, out_hbm, idx_v, g_v, t_v, *sems):
        sem_g, sem_s = sems[:nb], sems[nb:]
        wid = lax.axis_index("s") * nc + lax.axis_index("c")
        base = wid * per_w
        iota = lax.iota(jnp.int32, nl)

        def issue_gather(i, b):
            off = pl.multiple_of(base + i * rows, 8)
            pltpu.sync_copy(idx_hbm.at[pl.ds(off, rows)], idx_v.at[b])
            pltpu.async_copy(table_hbm.at[idx_v.at[b]], g_v.at[b], sem_g[b])

        def wait_gather(b):
            pltpu.make_async_copy(
                table_hbm.at[idx_v.at[b]], g_v.at[b], sem_g[b]).wait()

        def out_slice(i):
            ct = wid * 4 + i // (hh // hb)
            h0 = (i % (hh // hb)) * hb
            return out_hbm.at[pl.ds(h0, hb), :, ct]

        def issue_store(i, b):
            pltpu.async_copy(t_v.at[b], out_slice(i), sem_s[b])

        def wait_store(i, b):
            pltpu.make_async_copy(t_v.at[b], out_slice(i), sem_s[b]).wait()

        def transpose(b):
            g2, t2 = g_v.at[b], t_v.at[b]

            @plsc.parallel_loop(0, 128 // nl, unroll=2)
            def _(k):
                c0 = pl.multiple_of(k * nl, nl)
                cv = iota + c0
                for d0 in range(d):
                    dv = (iota + d0) & (d - 1)
                    rtv = dv >> 3
                    rv = dv & 7
                    for hp in range(hb):
                        rvec = cv + hp * 128
                        vec = plsc.load_gather(g2, [rvec, dv])
                        plsc.store_scatter(t2.at[hp], [rtv, rv, cv], vec)

        for b in range(nb):
            issue_gather(b, b)

        def outer(jo, carry):
            for b in range(nb):
                i = jo * nb + b
                wait_gather(b)
                transpose(b)
                issue_store(i, b)
                wait_store(i, b)
                issue_gather(i + nb, b)
            return carry

        lax.fori_loop(0, cpw // nb - 1, outer, 0, unroll=False)

        for b in range(nb):
            i = cpw - nb + b
            wait_gather(b)
            transpose(b)
            issue_store(i, b)
            wait_store(i, b)

    return gather_kernel


def kernel(x, table):
    b, h = x.shape
    _, d = table.shape
    nw = 32
    hb = _HB
    xp = (x.astype(jnp.int32)
          .reshape(nw, 4, 128, h // hb, hb)
          .transpose(0, 1, 3, 4, 2)
          .reshape(b * h))
    u = _make_gather(b, h, d)(xp, table)
    return u.transpose(2, 4, 0, 1, 3).reshape(b, h, d)


# speedup vs baseline: 2.1163x; 1.0145x over previous
import functools

import jax
import jax.numpy as jnp
from jax import lax
from jax.experimental import pallas as pl
from jax.experimental.pallas import tpu as pltpu
from jax.experimental.pallas import tpu_sc as plsc

_HB = 4
_NB = 2


@functools.lru_cache(maxsize=None)
def _make_gather(bb, hh, d):
    info = plsc.get_sparse_core_info()
    nc, ns, nl = info.num_cores, info.num_subcores, info.num_lanes
    nw = nc * ns
    n = bb * hh
    hb = _HB
    nb = _NB
    cpw = 4 * (hh // hb)
    rows = hb * 128
    per_w = n // nw
    assert bb == nw * 4 * 128 and hh % hb == 0 and d % 8 == 0
    assert 8 % hb == 0 or hb % 8 == 0

    mesh = plsc.VectorSubcoreMesh(core_axis_name="c", subcore_axis_name="s")

    @functools.partial(
        pl.kernel,
        mesh=mesh,
        out_type=jax.ShapeDtypeStruct((hh, d // 8, bb // 128, 8, 128),
                                      jnp.float32),
        compiler_params=pltpu.CompilerParams(use_tc_tiling_on_sc=False,
                                             needs_layout_passes=False),
        scratch_types=[
            pltpu.VMEM((nb, hb, 128), jnp.int32),
            pltpu.VMEM((nb, rows, d), jnp.float32),
            pltpu.VMEM((nb, hb, d // 8, 8, 128), jnp.float32),
        ] + [pltpu.SemaphoreType.DMA] * (2 * nb),
    )
    def gather_kernel(idx_hbm, table_hbm, out_hbm, idx_v, g_v, t_v, *sems):
        sem_g, sem_s = sems[:nb], sems[nb:]
        wid = lax.axis_index("s") * nc + lax.axis_index("c")
        base = wid * per_w
        iota = lax.iota(jnp.int32, nl)

        def issue_gather(i, b):
            ct = wid * 4 + i // (hh // hb)
            h0 = (i % (hh // hb)) * hb
            pltpu.sync_copy(
                idx_hbm.at[h0 // 8, ct, pl.ds(h0 % 8, hb)], idx_v.at[b])
            for hp in range(hb):
                pltpu.async_copy(table_hbm.at[idx_v.at[b, hp]],
                                 g_v.at[b, pl.ds(hp * 128, 128)], sem_g[b])

        def wait_gather(b):
            for hp in range(hb):
                pltpu.make_async_copy(
                    table_hbm.at[idx_v.at[b, hp]],
                    g_v.at[b, pl.ds(hp * 128, 128)], sem_g[b]).wait()

        def out_slice(i):
            ct = wid * 4 + i // (hh // hb)
            h0 = (i % (hh // hb)) * hb
            return out_hbm.at[pl.ds(h0, hb), :, ct]

        def issue_store(i, b):
            pltpu.async_copy(t_v.at[b], out_slice(i), sem_s[b])

        def wait_store(i, b):
            pltpu.make_async_copy(t_v.at[b], out_slice(i), sem_s[b]).wait()

        def transpose(b):
            g2, t2 = g_v.at[b], t_v.at[b]

            @plsc.parallel_loop(0, 128 // nl, unroll=2)
            def _(k):
                c0 = pl.multiple_of(k * nl, nl)
                cv = iota + c0
                for d0 in range(d):
                    dv = (iota + d0) & (d - 1)
                    rtv = dv >> 3
                    rv = dv & 7
                    for hp in range(hb):
                        rvec = cv + hp * 128
                        vec = plsc.load_gather(g2, [rvec, dv])
                        plsc.store_scatter(t2.at[hp], [rtv, rv, cv], vec)

        for b in range(nb):
            issue_gather(b, b)

        def outer(jo, carry):
            for b in range(nb):
                i = jo * nb + b
                wait_gather(b)
                transpose(b)
                issue_store(i, b)
                wait_store(i, b)
                issue_gather(i + nb, b)
            return carry

        lax.fori_loop(0, cpw // nb - 1, outer, 0, unroll=False)

        for b in range(nb):
            i = cpw - nb + b
            wait_gather(b)
            transpose(b)
            issue_store(i, b)
            wait_store(i, b)

    return gather_kernel


def kernel(x, table):
    b, h = x.shape
    _, d = table.shape
    xq = (x.astype(jnp.int32)
          .T.reshape(h // 8, 8, b // 128, 128)
          .transpose(0, 2, 1, 3))
    u = _make_gather(b, h, d)(xq, table)
    return u.transpose(2, 4, 0, 1, 3).reshape(b, h, d)
